# baseline (speedup 1.0000x reference)
import jax
import jax.numpy as jnp
from jax import lax
from jax.experimental import pallas as pl
from jax.experimental.pallas import tpu as pltpu
from jax.experimental.pallas import tpu_sc as plsc

F = 26
B = 4096
V = 100000
D = 64
DP = 128
NLANE = 16
NC = 2
NS = 16
NW = NC * NS
NB = 128
NBLK = B // NB
NPAIR = F // 2
NITEMS = NPAIR * NBLK
IPW = NITEMS // NW
C = 384
CJ = C // 128
KD = D // NLANE


def _sload(ref, i):
    return ref[pl.ds(i, NLANE)][0]


def _sc_body(vals_hbm, offs_hbm, tab_hbm, out_hbm,
             offs_v, vals_v, idx_v, po_v, rows_v, out_v, sems):
    w = lax.axis_index("s") * NC + lax.axis_index("c")

    def item_loop(i, _):
        item = w * IPW + i
        g = item // NBLK
        blk = item % NBLK
        bag0 = blk * NB

        def zero_loop(z, _):
            for k in range(DP // NLANE):
                out_v[z, pl.ds(k * NLANE, NLANE)] = jnp.zeros((NLANE,),
                                                              jnp.float32)
            return 0

        lax.fori_loop(0, NB, zero_loop, 0)

        def phase(h):
            f = 2 * g + h
            goff = f * B + bag0
            col = h * D

            pltpu.sync_copy(offs_hbm.at[pl.ds(goff, NB + 32)], offs_v)

            p_start = _sload(offs_v, 0)
            p_end = _sload(offs_v, NB)
            p8 = (p_start // 8) * 8
            nsub = (p_end - p8 + (C - 1)) // C
            fV = f * V

            def fire(fsub, par):
                base = p8 + fsub * C
                pltpu.sync_copy(vals_hbm.at[pl.ds(base, C)], vals_v)
                for j in range(CJ):
                    for k in range(128 // NLANE):
                        flat = (vals_v[pl.ds(j * 128 + k * NLANE, NLANE)]
                                + fV)
                        idx_v[par, j, pl.ds(k * NLANE, NLANE)] = flat >> 1
                        po_v[pl.ds(par * C + j * 128 + k * NLANE, NLANE)] = (
                            (flat & 1) * D)
                for j in range(CJ):
                    pltpu.async_copy(
                        tab_hbm.at[idx_v.at[par, j]],
                        rows_v.at[par, pl.ds(j * 128, 128)], sems.at[par])

            @pl.when(nsub > 0)
            def _():
                fire(jnp.int32(0), jnp.int32(0))

            def sub_loop(sub, carry):
                s, a0, a1, a2, a3 = carry
                par = lax.rem(sub, 2)

                @pl.when(sub + 1 < nsub)
                def _():
                    fire(sub + 1, 1 - par)

                pltpu.make_async_copy(
                    tab_hbm.at[pl.ds(0, C)], rows_v.at[par],
                    sems.at[par]).wait()

                base = p8 + sub * C
                lim = base + C

                def bs_step(_, lh):
                    blo, bhi = lh
                    mid = (blo + bhi) // 2
                    pred = _sload(offs_v, mid) < lim
                    blo = jnp.where(pred, mid + 1, blo)
                    bhi = jnp.where(pred, bhi, mid)
                    return (blo, bhi)

                s_end, _ = lax.fori_loop(0, 8, bs_step, (s, jnp.int32(NB)))

                def bag_body(sb, a):
                    a0, a1, a2, a3 = a
                    o_pair = offs_v[pl.ds(sb, NLANE)]
                    o_lo = o_pair[0]
                    o_hi = o_pair[1]
                    lo = jnp.maximum(o_lo, base)
                    hi = jnp.minimum(o_hi, lim)

                    def row_body(r, aa):
                        lr = r - base
                        pb = po_v[pl.ds(par * C + lr, NLANE)][0]
                        return tuple(
                            aa[k] + rows_v[par, lr,
                                           pl.ds(pl.multiple_of(
                                               pb + k * NLANE, NLANE),
                                               NLANE)]
                            for k in range(KD))

                    a0, a1, a2, a3 = lax.fori_loop(lo, hi, row_body,
                                                   (a0, a1, a2, a3))
                    done = o_hi <= lim
                    srow = jnp.where(done, sb, jnp.int32(NB))
                    out_v[srow, pl.ds(col, NLANE)] = a0
                    out_v[srow, pl.ds(col + NLANE, NLANE)] = a1
                    out_v[srow, pl.ds(col + 2 * NLANE, NLANE)] = a2
                    out_v[srow, pl.ds(col + 3 * NLANE, NLANE)] = a3
                    zero = jnp.zeros((NLANE,), jnp.float32)
                    a0 = jnp.where(done, zero, a0)
                    a1 = jnp.where(done, zero, a1)
                    a2 = jnp.where(done, zero, a2)
                    a3 = jnp.where(done, zero, a3)
                    return (a0, a1, a2, a3)

                a0, a1, a2, a3 = lax.fori_loop(s, s_end, bag_body,
                                               (a0, a1, a2, a3))
                last = jnp.maximum(s_end - 1, s)
                incomplete = _sload(offs_v, last + 1) > lim
                s = jnp.where(s_end > s,
                              s_end - incomplete.astype(jnp.int32), s)
                return (s, a0, a1, a2, a3)

            zero = jnp.zeros((NLANE,), jnp.float32)
            lax.fori_loop(0, nsub, sub_loop,
                          (jnp.int32(0), zero, zero, zero, zero))

        phase(0)
        phase(1)

        pltpu.sync_copy(out_v.at[pl.ds(0, NB)],
                        out_hbm.at[pl.ds(bag0, NB), pl.ds(g * DP, DP)])
        return 0

    lax.fori_loop(0, IPW, item_loop, 0)


@jax.jit
def _ebc_sc(vals_pad, offs_pad, tab_pad):
    mesh = plsc.VectorSubcoreMesh(core_axis_name="c", subcore_axis_name="s")
    return pl.kernel(
        _sc_body,
        out_type=jax.ShapeDtypeStruct((B, F * D), jnp.float32),
        mesh=mesh,
        compiler_params=pltpu.CompilerParams(use_tc_tiling_on_sc=True),
        scratch_types=[
            pltpu.VMEM((NB + 32,), jnp.int32),
            pltpu.VMEM((C,), jnp.int32),
            pltpu.VMEM((2, CJ, 128), jnp.int32),
            pltpu.VMEM((2 * C + NLANE,), jnp.int32),
            pltpu.VMEM((2, C, DP), jnp.float32),
            pltpu.VMEM((NB + 1, DP), jnp.float32),
            pltpu.SemaphoreType.DMA((2,)),
        ],
    )(vals_pad, offs_pad, tab_pad)


def kernel(values, offsets, tables):
    total = values.shape[0]
    vals_pad = jnp.concatenate(
        [values, jnp.zeros((C + 8,), jnp.int32)])
    offs_pad = jnp.concatenate(
        [offsets, jnp.full((NB + 64,), jnp.int32(total))])
    tab2 = tables.reshape(F * V // 2, DP)
    return _ebc_sc(vals_pad, offs_pad, tab2)
